# baseline (speedup 1.0000x reference)
import functools

import jax
import jax.numpy as jnp
from jax import lax
from jax.experimental import pallas as pl
from jax.experimental.pallas import tpu as pltpu
from jax.experimental.pallas import tpu_sc as plsc

N_NODES = 10000
N_EDGES = 320000
NC = 2
NS = 16
NW = NC * NS
EPW = N_EDGES // NW
CHUNK = 100
NCHUNK = EPW // CHUNK
ROWS_PER_TILE = 624
TAIL_ROWS = N_NODES - NS * ROWS_PER_TILE
TAIL_OFF = NS * ROWS_PER_TILE

BM = 1000


def _sc_segment_sum(msg, src, dst, zeros, d):
    mesh = plsc.VectorSubcoreMesh(
        core_axis_name="c", subcore_axis_name="s", num_cores=NC, num_subcores=NS
    )

    @functools.partial(
        pl.kernel,
        out_type=jax.ShapeDtypeStruct((NC, N_NODES, d), jnp.float32),
        mesh=mesh,
        scratch_types=[
            pltpu.VMEM((NCHUNK, CHUNK), jnp.int32),
            pltpu.VMEM((NCHUNK, CHUNK), jnp.int32),
            pltpu.VMEM((CHUNK, d), jnp.float32),
            pltpu.VMEM((CHUNK, d), jnp.float32),
            pltpu.VMEM_SHARED((N_NODES, d), jnp.float32),
            pltpu.SemaphoreType.DMA,
            pltpu.SemaphoreType.DMA,
        ],
        compiler_params=pltpu.CompilerParams(use_tc_tiling_on_sc=False),
    )
    def seg_kernel(msg_hbm, src_hbm, dst_hbm, zeros_hbm, out_hbm, src_v, dst_v,
                   rows0_v, rows1_v, acc_sh, sem, sem_s):
        c = lax.axis_index("c")
        s = lax.axis_index("s")
        w = c * NS + s
        r0 = s * ROWS_PER_TILE
        rows = (rows0_v, rows1_v)

        pltpu.sync_copy(src_hbm.at[w], src_v)
        pltpu.sync_copy(dst_hbm.at[w], dst_v)

        pltpu.sync_copy(
            zeros_hbm.at[pl.ds(r0, ROWS_PER_TILE)],
            acc_sh.at[pl.ds(r0, ROWS_PER_TILE)],
        )

        @pl.when(s == NS - 1)
        def _():
            pltpu.sync_copy(
                zeros_hbm.at[pl.ds(TAIL_OFF, TAIL_ROWS)],
                acc_sh.at[pl.ds(TAIL_OFF, TAIL_ROWS)],
            )

        plsc.subcore_barrier()

        pltpu.async_copy(msg_hbm.at[src_v.at[0]], rows0_v, sem)

        def body(i, carry):
            for b in range(2):
                j = 2 * i + b
                rb, ro = rows[b], rows[1 - b]
                pltpu.make_async_copy(msg_hbm.at[src_v.at[j]], rb, sem).wait()
                pltpu.async_copy(rb, acc_sh.at[dst_v.at[j]], sem_s, add=True)
                jm = jnp.maximum(j - 1, 0)

                @pl.when(j >= 1)
                def _():
                    pltpu.make_async_copy(
                        ro, acc_sh.at[dst_v.at[jm]], sem_s).wait()

                jn = jnp.minimum(j + 1, NCHUNK - 1)

                @pl.when(j + 1 < NCHUNK)
                def _():
                    pltpu.async_copy(msg_hbm.at[src_v.at[jn]], ro, sem)
            return carry

        lax.fori_loop(0, NCHUNK // 2, body, 0)
        pltpu.make_async_copy(
            rows[(NCHUNK - 1) % 2],
            acc_sh.at[dst_v.at[NCHUNK - 1]], sem_s).wait()
        plsc.subcore_barrier()
        pltpu.sync_copy(
            acc_sh.at[pl.ds(r0, ROWS_PER_TILE)],
            out_hbm.at[c, pl.ds(r0, ROWS_PER_TILE)],
        )

        @pl.when(s == NS - 1)
        def _():
            pltpu.sync_copy(
                acc_sh.at[pl.ds(TAIL_OFF, TAIL_ROWS)],
                out_hbm.at[c, pl.ds(TAIL_OFF, TAIL_ROWS)],
            )

    return seg_kernel(msg, src, dst, zeros)


def _mm_body(x_ref, w_ref, o_ref):
    o_ref[...] = jnp.dot(x_ref[...], w_ref[...],
                         preferred_element_type=jnp.float32)


def _matmul(x, w):
    m, k = x.shape
    n = w.shape[1]
    return pl.pallas_call(
        _mm_body,
        grid=(m // BM,),
        in_specs=[
            pl.BlockSpec((BM, k), lambda i: (i, 0)),
            pl.BlockSpec((k, n), lambda i: (0, 0)),
        ],
        out_specs=pl.BlockSpec((BM, n), lambda i: (i, 0)),
        out_shape=jax.ShapeDtypeStruct((m, n), jnp.float32),
    )(x, w)


def _relu_mm_body(p_ref, b_ref, w_ref, o_ref):
    h = jnp.maximum(p_ref[0] + p_ref[1] + b_ref[...], 0.0)
    o_ref[...] = jnp.dot(h, w_ref[...], preferred_element_type=jnp.float32)


def _relu_matmul(partials, b, w):
    _, m, k = partials.shape
    n = w.shape[1]
    return pl.pallas_call(
        _relu_mm_body,
        grid=(m // BM,),
        in_specs=[
            pl.BlockSpec((2, BM, k), lambda i: (0, i, 0)),
            pl.BlockSpec((1, k), lambda i: (0, 0)),
            pl.BlockSpec((k, n), lambda i: (0, 0)),
        ],
        out_specs=pl.BlockSpec((BM, n), lambda i: (i, 0)),
        out_shape=jax.ShapeDtypeStruct((m, n), jnp.float32),
    )(partials, b.reshape(1, k), w)


def _decoder_body(p_ref, b_ref, eps_ref, o_ref):
    i = pl.program_id(0)
    z = p_ref[0] + p_ref[1] + b_ref[...]
    z = z + jnp.sqrt(jnp.exp(z)) * eps_ref[...]
    g = lax.dot_general(z, z, (((0,), (0,)), ((), ())),
                        preferred_element_type=jnp.float32)

    @pl.when(i == 0)
    def _():
        o_ref[...] = g

    @pl.when(i > 0)
    def _():
        o_ref[...] += g

    @pl.when(i == pl.num_programs(0) - 1)
    def _():
        o_ref[...] = jax.nn.sigmoid(o_ref[...])


def _decoder(partials, b, eps):
    _, m, dz = partials.shape
    return pl.pallas_call(
        _decoder_body,
        grid=(m // BM,),
        in_specs=[
            pl.BlockSpec((2, BM, dz), lambda i: (0, i, 0)),
            pl.BlockSpec((1, dz), lambda i: (0, 0)),
            pl.BlockSpec((BM, dz), lambda i: (i, 0)),
        ],
        out_specs=pl.BlockSpec((dz, dz), lambda i: (0, 0)),
        out_shape=jax.ShapeDtypeStruct((dz, dz), jnp.float32),
    )(partials, b.reshape(1, dz), eps)


def kernel(X, edge_index, W1, b1, W2, b2, eps):
    d_h = W1.shape[1]
    d_z = W2.shape[1]
    zeros_h = jnp.zeros((N_NODES, d_h), jnp.float32)
    zeros_z = jnp.zeros((N_NODES, d_z), jnp.float32)
    src = edge_index[0].reshape(NW, NCHUNK, CHUNK)
    dst = edge_index[1].reshape(NW, NCHUNK, CHUNK)

    msg1 = _matmul(X, W1)
    part1 = _sc_segment_sum(msg1, src, dst, zeros_h, d_h)
    msg2 = _relu_matmul(part1, b1, W2)
    part2 = _sc_segment_sum(msg2, src, dst, zeros_z, d_z)
    G = _decoder(part2, b2, eps)
    return G.reshape(-1)


# speedup vs baseline: 1.0584x; 1.0584x over previous
import functools

import jax
import jax.numpy as jnp
from jax import lax
from jax.experimental import pallas as pl
from jax.experimental.pallas import tpu as pltpu
from jax.experimental.pallas import tpu_sc as plsc

N_NODES = 10000
N_EDGES = 320000
NC = 2
NS = 16
NW = NC * NS
EPW = N_EDGES // NW
CHUNK = 50
NCHUNK = EPW // CHUNK
ROWS_PER_TILE = 624
TAIL_ROWS = N_NODES - NS * ROWS_PER_TILE
TAIL_OFF = NS * ROWS_PER_TILE

BM = 1000


def _sc_segment_sum(msg, src, dst, zeros, d):
    mesh = plsc.VectorSubcoreMesh(
        core_axis_name="c", subcore_axis_name="s", num_cores=NC, num_subcores=NS
    )

    @functools.partial(
        pl.kernel,
        out_type=jax.ShapeDtypeStruct((NC, N_NODES, d), jnp.float32),
        mesh=mesh,
        scratch_types=[
            pltpu.VMEM((NCHUNK, CHUNK), jnp.int32),
            pltpu.VMEM((NCHUNK, CHUNK), jnp.int32),
            pltpu.VMEM((CHUNK, d), jnp.float32),
            pltpu.VMEM((CHUNK, d), jnp.float32),
            pltpu.VMEM((CHUNK, d), jnp.float32),
            pltpu.VMEM((CHUNK, d), jnp.float32),
            pltpu.VMEM_SHARED((N_NODES, d), jnp.float32),
            pltpu.SemaphoreType.DMA,
            pltpu.SemaphoreType.DMA,
        ],
        compiler_params=pltpu.CompilerParams(use_tc_tiling_on_sc=False),
    )
    def seg_kernel(msg_hbm, src_hbm, dst_hbm, zeros_hbm, out_hbm, src_v, dst_v,
                   rows0_v, rows1_v, rows2_v, rows3_v, acc_sh, sem, sem_s):
        c = lax.axis_index("c")
        s = lax.axis_index("s")
        w = c * NS + s
        r0 = s * ROWS_PER_TILE
        rows = (rows0_v, rows1_v, rows2_v, rows3_v)

        pltpu.sync_copy(src_hbm.at[w], src_v)
        pltpu.sync_copy(dst_hbm.at[w], dst_v)

        pltpu.sync_copy(
            zeros_hbm.at[pl.ds(r0, ROWS_PER_TILE)],
            acc_sh.at[pl.ds(r0, ROWS_PER_TILE)],
        )

        @pl.when(s == NS - 1)
        def _():
            pltpu.sync_copy(
                zeros_hbm.at[pl.ds(TAIL_OFF, TAIL_ROWS)],
                acc_sh.at[pl.ds(TAIL_OFF, TAIL_ROWS)],
            )

        plsc.subcore_barrier()

        pltpu.async_copy(msg_hbm.at[src_v.at[0]], rows[0], sem)
        pltpu.async_copy(msg_hbm.at[src_v.at[1]], rows[1], sem)

        def body(i, carry):
            for k in range(4):
                j = 4 * i + k
                rb = rows[k]
                rn = rows[(k + 2) % 4]
                pltpu.make_async_copy(msg_hbm.at[src_v.at[j]], rb, sem).wait()
                pltpu.async_copy(rb, acc_sh.at[dst_v.at[j]], sem_s, add=True)
                jm = jnp.maximum(j - 2, 0)

                @pl.when(j >= 2)
                def _():
                    pltpu.make_async_copy(
                        rn, acc_sh.at[dst_v.at[jm]], sem_s).wait()

                jn = jnp.minimum(j + 2, NCHUNK - 1)

                @pl.when(j + 2 < NCHUNK)
                def _():
                    pltpu.async_copy(msg_hbm.at[src_v.at[jn]], rn, sem)
            return carry

        lax.fori_loop(0, NCHUNK // 4, body, 0)
        pltpu.make_async_copy(
            rows[(NCHUNK - 2) % 4],
            acc_sh.at[dst_v.at[NCHUNK - 2]], sem_s).wait()
        pltpu.make_async_copy(
            rows[(NCHUNK - 1) % 4],
            acc_sh.at[dst_v.at[NCHUNK - 1]], sem_s).wait()
        plsc.subcore_barrier()
        pltpu.sync_copy(
            acc_sh.at[pl.ds(r0, ROWS_PER_TILE)],
            out_hbm.at[c, pl.ds(r0, ROWS_PER_TILE)],
        )

        @pl.when(s == NS - 1)
        def _():
            pltpu.sync_copy(
                acc_sh.at[pl.ds(TAIL_OFF, TAIL_ROWS)],
                out_hbm.at[c, pl.ds(TAIL_OFF, TAIL_ROWS)],
            )

    return seg_kernel(msg, src, dst, zeros)


def _mm_body(x_ref, w_ref, o_ref):
    o_ref[...] = jnp.dot(x_ref[...], w_ref[...],
                         preferred_element_type=jnp.float32)


def _matmul(x, w):
    m, k = x.shape
    n = w.shape[1]
    return pl.pallas_call(
        _mm_body,
        grid=(m // BM,),
        in_specs=[
            pl.BlockSpec((BM, k), lambda i: (i, 0)),
            pl.BlockSpec((k, n), lambda i: (0, 0)),
        ],
        out_specs=pl.BlockSpec((BM, n), lambda i: (i, 0)),
        out_shape=jax.ShapeDtypeStruct((m, n), jnp.float32),
    )(x, w)


def _relu_mm_body(p_ref, b_ref, w_ref, o_ref):
    h = jnp.maximum(p_ref[0] + p_ref[1] + b_ref[...], 0.0)
    o_ref[...] = jnp.dot(h, w_ref[...], preferred_element_type=jnp.float32)


def _relu_matmul(partials, b, w):
    _, m, k = partials.shape
    n = w.shape[1]
    return pl.pallas_call(
        _relu_mm_body,
        grid=(m // BM,),
        in_specs=[
            pl.BlockSpec((2, BM, k), lambda i: (0, i, 0)),
            pl.BlockSpec((1, k), lambda i: (0, 0)),
            pl.BlockSpec((k, n), lambda i: (0, 0)),
        ],
        out_specs=pl.BlockSpec((BM, n), lambda i: (i, 0)),
        out_shape=jax.ShapeDtypeStruct((m, n), jnp.float32),
    )(partials, b.reshape(1, k), w)


def _decoder_body(p_ref, b_ref, eps_ref, o_ref):
    i = pl.program_id(0)
    z = p_ref[0] + p_ref[1] + b_ref[...]
    z = z + jnp.sqrt(jnp.exp(z)) * eps_ref[...]
    g = lax.dot_general(z, z, (((0,), (0,)), ((), ())),
                        preferred_element_type=jnp.float32)

    @pl.when(i == 0)
    def _():
        o_ref[...] = g

    @pl.when(i > 0)
    def _():
        o_ref[...] += g

    @pl.when(i == pl.num_programs(0) - 1)
    def _():
        o_ref[...] = jax.nn.sigmoid(o_ref[...])


def _decoder(partials, b, eps):
    _, m, dz = partials.shape
    return pl.pallas_call(
        _decoder_body,
        grid=(m // BM,),
        in_specs=[
            pl.BlockSpec((2, BM, dz), lambda i: (0, i, 0)),
            pl.BlockSpec((1, dz), lambda i: (0, 0)),
            pl.BlockSpec((BM, dz), lambda i: (i, 0)),
        ],
        out_specs=pl.BlockSpec((dz, dz), lambda i: (0, 0)),
        out_shape=jax.ShapeDtypeStruct((dz, dz), jnp.float32),
    )(partials, b.reshape(1, dz), eps)


def kernel(X, edge_index, W1, b1, W2, b2, eps):
    d_h = W1.shape[1]
    d_z = W2.shape[1]
    zeros_h = jnp.zeros((N_NODES, d_h), jnp.float32)
    zeros_z = jnp.zeros((N_NODES, d_z), jnp.float32)
    src = edge_index[0].reshape(NW, NCHUNK, CHUNK)
    dst = edge_index[1].reshape(NW, NCHUNK, CHUNK)

    msg1 = _matmul(X, W1)
    part1 = _sc_segment_sum(msg1, src, dst, zeros_h, d_h)
    msg2 = _relu_matmul(part1, b1, W2)
    part2 = _sc_segment_sum(msg2, src, dst, zeros_z, d_z)
    G = _decoder(part2, b2, eps)
    return G.reshape(-1)
